# Initial kernel scaffold; baseline (speedup 1.0000x reference)
import jax
import jax.numpy as jnp
from jax.experimental import pallas as pl


def kernel(h, edge_index, W, a):
    raise NotImplementedError("write your pallas kernel here")


# speedup vs baseline: 10.8789x; 10.8789x over previous
import functools

import jax
import jax.numpy as jnp
from jax import lax
from jax.experimental import pallas as pl
from jax.experimental.pallas import tpu as pltpu
from jax.experimental.pallas import tpu_sc as plsc

N = 10000
E = 320000
D = 128
NEG = 0.2

NC = 2
NS = 16
NW = NC * NS
C = 128
NCHUNK = E // C
KMAX = -(-NCHUNK // NW)
NPAD = 10240
PER_TILE = NPAD // NS


def _wh_body(h_ref, w_ref, wh_ref):
    wh_ref[...] = jnp.dot(h_ref[...], w_ref[...],
                          preferred_element_type=jnp.float32)


def _st_body(a_ref, wh_ref, st_ref):
    st_ref[...] = lax.dot_general(
        a_ref[...], wh_ref[...], (((1,), (1,)), ((), ())),
        preferred_element_type=jnp.float32)


def _dense(h, W, A):
    bn = 2000
    wh = pl.pallas_call(
        _wh_body,
        grid=(N // bn,),
        in_specs=[
            pl.BlockSpec((bn, D), lambda i: (i, 0)),
            pl.BlockSpec((D, D), lambda i: (0, 0)),
        ],
        out_specs=pl.BlockSpec((bn, D), lambda i: (i, 0)),
        out_shape=jax.ShapeDtypeStruct((N, D), jnp.float32),
    )(h, W)
    st = pl.pallas_call(
        _st_body,
        in_specs=[
            pl.BlockSpec((8, D), lambda: (0, 0)),
            pl.BlockSpec((N, D), lambda: (0, 0)),
        ],
        out_specs=pl.BlockSpec((8, N), lambda: (0, 0)),
        out_shape=jax.ShapeDtypeStruct((8, N), jnp.float32),
    )(A, wh)
    return wh, st


def _splat(vec_ref, r):
    return plsc.load_gather(vec_ref, [jnp.full((16,), r, jnp.int32)])


def _edge_body(wh_hbm, st_hbm, row_hbm, col_hbm, accp_hbm, denp_hbm,
               s_v, t_v, rowbuf, xbuf, ridx, cidx, acc_sh, den_sh, sem):
    cid = lax.axis_index("c")
    sid = lax.axis_index("s")
    wid = sid * NC + cid

    pltpu.sync_copy(st_hbm.at[0], s_v)
    pltpu.sync_copy(st_hbm.at[1], t_v)

    def _mbody(i, carry):
        ms, mt = carry
        off = pl.multiple_of(i * 16, 16)
        return (jnp.maximum(ms, s_v[pl.ds(off, 16)]),
                jnp.maximum(mt, t_v[pl.ds(off, 16)]))
    ms, mt = lax.fori_loop(0, N // 16, _mbody,
                           (jnp.full((16,), -jnp.inf, jnp.float32),
                            jnp.full((16,), -jnp.inf, jnp.float32)))
    xbuf[pl.ds(0, 16)] = plsc.cummax(ms)
    xbuf[pl.ds(16, 16)] = plsc.cummax(mt)
    m0 = _splat(xbuf, 15) + _splat(xbuf, 31)
    m = jnp.where(m0 > 0, m0, NEG * m0)

    def _zrow(r, _):
        for i in range(8):
            rowbuf[r, pl.ds(i * 16, 16)] = jnp.zeros((16,), jnp.float32)
        return 0
    lax.fori_loop(0, C, _zrow, 0)
    for i in range(8):
        xbuf[pl.ds(i * 16, 16)] = jnp.zeros((16,), jnp.float32)
    base_t = sid * PER_TILE
    def _zacc(j, _):
        off = base_t + j * C
        pltpu.sync_copy(rowbuf, acc_sh.at[pl.ds(off, C)])
        pltpu.sync_copy(xbuf, den_sh.at[pl.ds(off, C)])
        return 0
    lax.fori_loop(0, PER_TILE // C, _zacc, 0)
    plsc.subcore_barrier()

    def _ebody(k, _):
        q = wid + NW * k

        @pl.when(q < NCHUNK)
        def _():
            pltpu.sync_copy(row_hbm.at[q], ridx)
            pltpu.sync_copy(col_hbm.at[q], cidx)
            pltpu.async_copy(wh_hbm.at[cidx], rowbuf, sem).wait()
            for i in range(C // 16):
                off = i * 16
                rv = ridx[pl.ds(off, 16)]
                cv = cidx[pl.ds(off, 16)]
                e = plsc.load_gather(s_v, [rv]) + plsc.load_gather(t_v, [cv])
                e = jnp.where(e > 0, e, NEG * e) - m
                xbuf[pl.ds(off, 16)] = jnp.exp(e)

            def _rbody(r, _):
                xr = _splat(xbuf, r)
                for i in range(8):
                    sl = pl.ds(i * 16, 16)
                    rowbuf[r, sl] = rowbuf[r, sl] * xr
                return 0
            lax.fori_loop(0, C, _rbody, 0)
            pltpu.sync_copy(rowbuf, acc_sh.at[ridx], add=True)
            pltpu.sync_copy(xbuf, den_sh.at[ridx], add=True)
        return 0
    lax.fori_loop(0, KMAX, _ebody, 0)
    plsc.subcore_barrier()

    pltpu.sync_copy(acc_sh.at[pl.ds(base_t, PER_TILE)],
                    accp_hbm.at[cid, pl.ds(base_t, PER_TILE)])
    pltpu.sync_copy(den_sh.at[pl.ds(base_t, PER_TILE)],
                    denp_hbm.at[pl.ds(cid * NPAD + base_t, PER_TILE)])


def _edge_pass(wh, st, row2d, col2d):
    mesh = plsc.VectorSubcoreMesh(core_axis_name="c", subcore_axis_name="s")
    k = functools.partial(
        pl.kernel,
        out_type=[
            jax.ShapeDtypeStruct((NC, NPAD, D), jnp.float32),
            jax.ShapeDtypeStruct((NC * NPAD,), jnp.float32),
        ],
        mesh=mesh,
        scratch_types=[
            pltpu.VMEM((N,), jnp.float32),
            pltpu.VMEM((N,), jnp.float32),
            pltpu.VMEM((C, D), jnp.float32),
            pltpu.VMEM((C,), jnp.float32),
            pltpu.VMEM((C,), jnp.int32),
            pltpu.VMEM((C,), jnp.int32),
            pltpu.VMEM_SHARED((NPAD, D), jnp.float32),
            pltpu.VMEM_SHARED((NPAD,), jnp.float32),
            pltpu.SemaphoreType.DMA,
        ],
        compiler_params=pltpu.CompilerParams(needs_layout_passes=False),
    )(_edge_body)
    return k(wh, st, row2d, col2d)


def _combine_body(accp_hbm, denp_hbm, out_hbm, b0, b1, d0, d1):
    cid = lax.axis_index("c")
    sid = lax.axis_index("s")
    wid = sid * NC + cid
    rows = NPAD // NW
    bw = 80
    for j in range(rows // bw):
        base = wid * rows + j * bw
        pltpu.sync_copy(accp_hbm.at[0, pl.ds(base, bw)], b0)
        pltpu.sync_copy(accp_hbm.at[1, pl.ds(base, bw)], b1)
        pltpu.sync_copy(denp_hbm.at[pl.ds(base, bw)], d0)
        pltpu.sync_copy(denp_hbm.at[pl.ds(NPAD + base, bw)], d1)

        def _rbody(r, _):
            dr = 1.0 / (_splat(d0, r) + _splat(d1, r) + 1e-10)
            for i in range(8):
                sl = pl.ds(i * 16, 16)
                b0[r, sl] = (b0[r, sl] + b1[r, sl]) * dr
            return 0
        lax.fori_loop(0, bw, _rbody, 0)
        pltpu.sync_copy(b0, out_hbm.at[pl.ds(base, bw)])


def _combine(accp, denp):
    mesh = plsc.VectorSubcoreMesh(core_axis_name="c", subcore_axis_name="s")
    k = functools.partial(
        pl.kernel,
        out_type=jax.ShapeDtypeStruct((NPAD, D), jnp.float32),
        mesh=mesh,
        scratch_types=[
            pltpu.VMEM((80, D), jnp.float32),
            pltpu.VMEM((80, D), jnp.float32),
            pltpu.VMEM((80,), jnp.float32),
            pltpu.VMEM((80,), jnp.float32),
        ],
        compiler_params=pltpu.CompilerParams(needs_layout_passes=False),
    )(_combine_body)
    return k(accp, denp)


def kernel(h, edge_index, W, a):
    A = (jnp.zeros((8, D), jnp.float32)
         .at[0, :].set(a[:D, 0])
         .at[1, :].set(a[D:, 0]))
    wh, st = _dense(h, W, A)
    row2d = edge_index[0].reshape(NCHUNK, C)
    col2d = edge_index[1].reshape(NCHUNK, C)
    accp, denp = _edge_pass(wh, st, row2d, col2d)
    out = _combine(accp, denp)
    return out[:N]


# speedup vs baseline: 20.3104x; 1.8670x over previous
import functools

import jax
import jax.numpy as jnp
from jax import lax
from jax.experimental import pallas as pl
from jax.experimental.pallas import tpu as pltpu
from jax.experimental.pallas import tpu_sc as plsc

N = 10000
E = 320000
D = 128
NEG = 0.2

NC = 2
NS = 16
NW = NC * NS
C = 128
NCHUNK = E // C
KMAX = -(-NCHUNK // NW)
NPAD = 10240
PER_TILE = NPAD // NS


def _wh_body(h_ref, w_ref, wh_ref):
    wh_ref[...] = jnp.dot(h_ref[...], w_ref[...],
                          preferred_element_type=jnp.float32)


def _st_body(a_ref, wh_ref, st_ref, mx_ref):
    stv = lax.dot_general(
        a_ref[...], wh_ref[...], (((1,), (1,)), ((), ())),
        preferred_element_type=jnp.float32)
    st_ref[...] = stv
    mx_ref[...] = jnp.broadcast_to(jnp.max(stv, axis=1, keepdims=True), (8, D))


def _dense(h, W, A):
    bn = 2000
    wh = pl.pallas_call(
        _wh_body,
        grid=(N // bn,),
        in_specs=[
            pl.BlockSpec((bn, D), lambda i: (i, 0)),
            pl.BlockSpec((D, D), lambda i: (0, 0)),
        ],
        out_specs=pl.BlockSpec((bn, D), lambda i: (i, 0)),
        out_shape=jax.ShapeDtypeStruct((N, D), jnp.float32),
    )(h, W)
    st, mx = pl.pallas_call(
        _st_body,
        in_specs=[
            pl.BlockSpec((8, D), lambda: (0, 0)),
            pl.BlockSpec((N, D), lambda: (0, 0)),
        ],
        out_specs=[
            pl.BlockSpec((8, N), lambda: (0, 0)),
            pl.BlockSpec((8, D), lambda: (0, 0)),
        ],
        out_shape=[
            jax.ShapeDtypeStruct((8, N), jnp.float32),
            jax.ShapeDtypeStruct((8, D), jnp.float32),
        ],
    )(A, wh)
    return wh, st, mx


def _splat(vec_ref, r):
    return plsc.load_gather(vec_ref, [jnp.full((16,), r, jnp.int32)])


def _edge_body(wh_hbm, s_hbm, t_hbm, mx_hbm, ei_hbm, accp_hbm, denp_hbm,
               rb0, rb1, sb0, sb1, tb0, tb1, xb0, xb1, ix0, ix1,
               acc_sh, den_sh, sem0, sem1):
    cid = lax.axis_index("c")
    sid = lax.axis_index("s")
    wid = sid * NC + cid
    rbufs, sbufs, tbufs = (rb0, rb1), (sb0, sb1), (tb0, tb1)
    xbufs, ixbufs, sems = (xb0, xb1), (ix0, ix1), (sem0, sem1)

    pltpu.sync_copy(mx_hbm.at[0], sb0)
    pltpu.sync_copy(mx_hbm.at[1], tb0)
    m0 = sb0[pl.ds(0, 16)] + tb0[pl.ds(0, 16)]
    m = jnp.where(m0 > 0, m0, NEG * m0)

    @plsc.parallel_loop(0, C)
    def _zrow(r):
        for i in range(8):
            rb0[r, pl.ds(i * 16, 16)] = jnp.zeros((16,), jnp.float32)
    for i in range(8):
        xb0[pl.ds(i * 16, 16)] = jnp.zeros((16,), jnp.float32)
    base_t = sid * PER_TILE
    def _zacc(j, _):
        off = base_t + j * C
        pltpu.sync_copy(rb0, acc_sh.at[pl.ds(off, C)])
        pltpu.sync_copy(xb0, den_sh.at[pl.ds(off, C)])
        return 0
    lax.fori_loop(0, PER_TILE // C, _zacc, 0)
    plsc.subcore_barrier()

    def _prefetch(q, slot):
        pltpu.sync_copy(ei_hbm.at[q], ixbufs[slot])
        pltpu.async_copy(wh_hbm.at[ixbufs[slot].at[1]], rbufs[slot],
                         sems[slot])
        pltpu.async_copy(s_hbm.at[ixbufs[slot].at[0]], sbufs[slot],
                         sems[slot])
        pltpu.async_copy(t_hbm.at[ixbufs[slot].at[1]], tbufs[slot],
                         sems[slot])

    def _drain(slot):
        pltpu.make_async_copy(wh_hbm.at[ixbufs[slot].at[1]], rbufs[slot],
                              sems[slot]).wait()
        pltpu.make_async_copy(s_hbm.at[ixbufs[slot].at[0]], sbufs[slot],
                              sems[slot]).wait()
        pltpu.make_async_copy(t_hbm.at[ixbufs[slot].at[1]], tbufs[slot],
                              sems[slot]).wait()

    _prefetch(wid, 0)

    def _jbody(j, _):
        for b in range(2):
            k = 2 * j + b
            q = wid + NW * k
            qn = q + NW
            bn = b ^ 1

            @pl.when(qn < NCHUNK)
            def _():
                _prefetch(qn, bn)

            @pl.when(q < NCHUNK)
            def _():
                rb, xb = rbufs[b], xbufs[b]
                _drain(b)
                for i in range(C // 16):
                    off = i * 16
                    e = (sbufs[b][pl.ds(off, 16)]
                         + tbufs[b][pl.ds(off, 16)])
                    e = jnp.where(e > 0, e, NEG * e) - m
                    xb[pl.ds(off, 16)] = jnp.exp(e)

                @plsc.parallel_loop(0, C, unroll=4)
                def _scale(r):
                    xr = _splat(xb, r)
                    for i in range(8):
                        sl = pl.ds(i * 16, 16)
                        rb[r, sl] = rb[r, sl] * xr
                pltpu.sync_copy(rb, acc_sh.at[ixbufs[b].at[0]], add=True)
                pltpu.sync_copy(xb, den_sh.at[ixbufs[b].at[0]], add=True)
        return 0
    lax.fori_loop(0, (KMAX + 1) // 2, _jbody, 0)
    plsc.subcore_barrier()

    pltpu.sync_copy(acc_sh.at[pl.ds(base_t, PER_TILE)],
                    accp_hbm.at[cid, pl.ds(base_t, PER_TILE)])
    pltpu.sync_copy(den_sh.at[pl.ds(base_t, PER_TILE)],
                    denp_hbm.at[pl.ds(cid * NPAD + base_t, PER_TILE)])


def _edge_pass(wh, s, t, mx, ei3):
    mesh = plsc.VectorSubcoreMesh(core_axis_name="c", subcore_axis_name="s")
    k = functools.partial(
        pl.kernel,
        out_type=[
            jax.ShapeDtypeStruct((NC, NPAD, D), jnp.float32),
            jax.ShapeDtypeStruct((NC * NPAD,), jnp.float32),
        ],
        mesh=mesh,
        scratch_types=[
            pltpu.VMEM((C, D), jnp.float32),
            pltpu.VMEM((C, D), jnp.float32),
            pltpu.VMEM((C,), jnp.float32),
            pltpu.VMEM((C,), jnp.float32),
            pltpu.VMEM((C,), jnp.float32),
            pltpu.VMEM((C,), jnp.float32),
            pltpu.VMEM((C,), jnp.float32),
            pltpu.VMEM((C,), jnp.float32),
            pltpu.VMEM((2, C), jnp.int32),
            pltpu.VMEM((2, C), jnp.int32),
            pltpu.VMEM_SHARED((NPAD, D), jnp.float32),
            pltpu.VMEM_SHARED((NPAD,), jnp.float32),
            pltpu.SemaphoreType.DMA,
            pltpu.SemaphoreType.DMA,
        ],
        compiler_params=pltpu.CompilerParams(needs_layout_passes=False),
    )(_edge_body)
    return k(wh, s, t, mx, ei3)


def _combine_body(accp_hbm, denp_hbm, out_hbm, b0, b1, d0, d1):
    cid = lax.axis_index("c")
    sid = lax.axis_index("s")
    wid = sid * NC + cid
    rows = NPAD // NW
    bw = 80
    for j in range(rows // bw):
        base = wid * rows + j * bw
        pltpu.sync_copy(accp_hbm.at[0, pl.ds(base, bw)], b0)
        pltpu.sync_copy(accp_hbm.at[1, pl.ds(base, bw)], b1)
        pltpu.sync_copy(denp_hbm.at[pl.ds(base, bw)], d0)
        pltpu.sync_copy(denp_hbm.at[pl.ds(NPAD + base, bw)], d1)

        def _rbody(r, _):
            dr = 1.0 / (_splat(d0, r) + _splat(d1, r) + 1e-10)
            for i in range(8):
                sl = pl.ds(i * 16, 16)
                b0[r, sl] = (b0[r, sl] + b1[r, sl]) * dr
            return 0
        lax.fori_loop(0, bw, _rbody, 0)
        pltpu.sync_copy(b0, out_hbm.at[pl.ds(base, bw)])


def _combine(accp, denp):
    mesh = plsc.VectorSubcoreMesh(core_axis_name="c", subcore_axis_name="s")
    k = functools.partial(
        pl.kernel,
        out_type=jax.ShapeDtypeStruct((NPAD, D), jnp.float32),
        mesh=mesh,
        scratch_types=[
            pltpu.VMEM((80, D), jnp.float32),
            pltpu.VMEM((80, D), jnp.float32),
            pltpu.VMEM((80,), jnp.float32),
            pltpu.VMEM((80,), jnp.float32),
        ],
        compiler_params=pltpu.CompilerParams(needs_layout_passes=False),
    )(_combine_body)
    return k(accp, denp)


def kernel(h, edge_index, W, a):
    A = (jnp.zeros((8, D), jnp.float32)
         .at[0, :].set(a[:D, 0])
         .at[1, :].set(a[D:, 0]))
    wh, st, mx = _dense(h, W, A)
    ei3 = edge_index.reshape(2, NCHUNK, C).transpose(1, 0, 2)
    accp, denp = _edge_pass(wh, st[0], st[1], mx, ei3)
    out = _combine(accp, denp)
    return out[:N]


# speedup vs baseline: 20.3900x; 1.0039x over previous
import functools

import jax
import jax.numpy as jnp
from jax import lax
from jax.experimental import pallas as pl
from jax.experimental.pallas import tpu as pltpu
from jax.experimental.pallas import tpu_sc as plsc

N = 10000
E = 320000
D = 128
NEG = 0.2

NC = 2
NS = 16
NW = NC * NS
C = 128
NCHUNK = E // C
KMAX = 80
CPAD = 2576
NPAD = 10240
PER_TILE = NPAD // NS


def _wh_body(h_ref, w_ref, wh_ref):
    wh_ref[...] = jnp.dot(h_ref[...], w_ref[...],
                          preferred_element_type=jnp.float32)


def _st_body(a_ref, wh_ref, st_ref, mx_ref):
    stv = lax.dot_general(
        a_ref[...], wh_ref[...], (((1,), (1,)), ((), ())),
        preferred_element_type=jnp.float32)
    st_ref[...] = stv
    mx_ref[...] = jnp.broadcast_to(jnp.max(stv, axis=1, keepdims=True), (8, D))


def _dense(h, W, A):
    bn = 2000
    wh = pl.pallas_call(
        _wh_body,
        grid=(N // bn,),
        in_specs=[
            pl.BlockSpec((bn, D), lambda i: (i, 0)),
            pl.BlockSpec((D, D), lambda i: (0, 0)),
        ],
        out_specs=pl.BlockSpec((bn, D), lambda i: (i, 0)),
        out_shape=jax.ShapeDtypeStruct((N, D), jnp.float32),
    )(h, W)
    st, mx = pl.pallas_call(
        _st_body,
        in_specs=[
            pl.BlockSpec((8, D), lambda: (0, 0)),
            pl.BlockSpec((N, D), lambda: (0, 0)),
        ],
        out_specs=[
            pl.BlockSpec((8, N), lambda: (0, 0)),
            pl.BlockSpec((8, D), lambda: (0, 0)),
        ],
        out_shape=[
            jax.ShapeDtypeStruct((8, N), jnp.float32),
            jax.ShapeDtypeStruct((8, D), jnp.float32),
        ],
    )(A, wh)
    return wh, st, mx


def _splat(vec_ref, r):
    return plsc.load_gather(vec_ref, [jnp.full((16,), r, jnp.int32)])


IXB = 8
NBATCH = 10


def _edge_body(wh_hbm, s_hbm, t_hbm, mx_hbm, ei_hbm, accp_hbm, denp_hbm,
               rb0, rb1, sb0, sb1, tb0, tb1, xb0, xb1, ixa, ixb,
               acc_sh, den_sh, gsem0, gsem1, ssem0, ssem1):
    cid = lax.axis_index("c")
    sid = lax.axis_index("s")
    wid = sid * NC + cid
    cstart = wid * 80
    cn = jnp.where(wid < NW - 1, 80, NCHUNK - 80 * (NW - 1))
    rbufs, sbufs, tbufs = (rb0, rb1), (sb0, sb1), (tb0, tb1)
    xbufs, ixs = (xb0, xb1), (ixa, ixb)
    gsems, ssems = (gsem0, gsem1), (ssem0, ssem1)

    pltpu.sync_copy(mx_hbm.at[0], sb0)
    pltpu.sync_copy(mx_hbm.at[1], tb0)
    m0 = sb0[pl.ds(0, 16)] + tb0[pl.ds(0, 16)]
    m = jnp.where(m0 > 0, m0, NEG * m0)

    @plsc.parallel_loop(0, C)
    def _zrow(r):
        for i in range(8):
            rb0[r, pl.ds(i * 16, 16)] = jnp.zeros((16,), jnp.float32)
    for i in range(8):
        xb0[pl.ds(i * 16, 16)] = jnp.zeros((16,), jnp.float32)
    base_t = sid * PER_TILE
    def _zacc(j, _):
        off = base_t + j * C
        pltpu.sync_copy(rb0, acc_sh.at[pl.ds(off, C)])
        pltpu.sync_copy(xb0, den_sh.at[pl.ds(off, C)])
        return 0
    lax.fori_loop(0, PER_TILE // C, _zacc, 0)
    plsc.subcore_barrier()

    def _gather(ix, kb, slot):
        pltpu.async_copy(wh_hbm.at[ix.at[2 * kb + 1]], rbufs[slot],
                         gsems[slot])
        pltpu.async_copy(s_hbm.at[ix.at[2 * kb]], sbufs[slot], gsems[slot])
        pltpu.async_copy(t_hbm.at[ix.at[2 * kb + 1]], tbufs[slot],
                         gsems[slot])

    def _gwait(ix, kb, slot):
        pltpu.make_async_copy(wh_hbm.at[ix.at[2 * kb + 1]], rbufs[slot],
                              gsems[slot]).wait()
        pltpu.make_async_copy(s_hbm.at[ix.at[2 * kb]], sbufs[slot],
                              gsems[slot]).wait()
        pltpu.make_async_copy(t_hbm.at[ix.at[2 * kb + 1]], tbufs[slot],
                              gsems[slot]).wait()

    def _swait(ix, kb, slot):
        pltpu.make_async_copy(rbufs[slot], acc_sh.at[ix.at[2 * kb]],
                              ssems[slot]).wait()
        pltpu.make_async_copy(xbufs[slot], den_sh.at[ix.at[2 * kb]],
                              ssems[slot]).wait()

    pltpu.sync_copy(ei_hbm.at[pl.ds(2 * cstart, 2 * IXB)], ixa)
    _gather(ixa, 0, 0)

    def _gbody(gp, _):
        for gg in range(2):
            g = 2 * gp + gg
            ix = ixs[gg]
            for kb in range(IXB):
                k_s = 16 * gp + 8 * gg + kb
                k = g * IXB + kb
                b = kb % 2
                bn = b ^ 1
                if kb == 4:
                    pltpu.sync_copy(
                        ei_hbm.at[pl.ds(2 * (cstart + (g + 1) * IXB),
                                        2 * IXB)],
                        ixs[gg ^ 1])

                @pl.when(k < cn)
                def _():
                    xb, rb = xbufs[b], rbufs[b]
                    _gwait(ix, kb, b)
                    for i in range(C // 16):
                        off = i * 16
                        e = (sbufs[b][pl.ds(off, 16)]
                             + tbufs[b][pl.ds(off, 16)])
                        e = jnp.where(e > 0, e, NEG * e) - m
                        xb[pl.ds(off, 16)] = jnp.exp(e)

                    @pl.when((k + 1 < cn) & (k >= 1))
                    def _():
                        if kb >= 1:
                            _swait(ix, kb - 1, bn)
                        else:
                            _swait(ixs[gg ^ 1], IXB - 1, bn)

                    @pl.when(k + 1 < cn)
                    def _():
                        if kb + 1 < IXB:
                            _gather(ix, kb + 1, bn)
                        else:
                            _gather(ixs[gg ^ 1], 0, bn)

                    @plsc.parallel_loop(0, C, unroll=4)
                    def _scale(r):
                        xr = _splat(xb, r)
                        for i in range(8):
                            sl = pl.ds(i * 16, 16)
                            rb[r, sl] = rb[r, sl] * xr
                    pltpu.async_copy(rb, acc_sh.at[ix.at[2 * kb]], ssems[b],
                                     add=True)
                    pltpu.async_copy(xb, den_sh.at[ix.at[2 * kb]], ssems[b],
                                     add=True)
        return 0
    lax.fori_loop(0, NBATCH // 2, _gbody, 0)
    _swait(ixa, 0, 0)
    _swait(ixa, 0, 1)
    plsc.subcore_barrier()

    pltpu.sync_copy(acc_sh.at[pl.ds(base_t, PER_TILE)],
                    accp_hbm.at[cid, pl.ds(base_t, PER_TILE)])
    pltpu.sync_copy(den_sh.at[pl.ds(base_t, PER_TILE)],
                    denp_hbm.at[pl.ds(cid * NPAD + base_t, PER_TILE)])


def _edge_pass(wh, s, t, mx, ei3):
    mesh = plsc.VectorSubcoreMesh(core_axis_name="c", subcore_axis_name="s")
    k = functools.partial(
        pl.kernel,
        out_type=[
            jax.ShapeDtypeStruct((NC, NPAD, D), jnp.float32),
            jax.ShapeDtypeStruct((NC * NPAD,), jnp.float32),
        ],
        mesh=mesh,
        scratch_types=[
            pltpu.VMEM((C, D), jnp.float32),
            pltpu.VMEM((C, D), jnp.float32),
            pltpu.VMEM((C,), jnp.float32),
            pltpu.VMEM((C,), jnp.float32),
            pltpu.VMEM((C,), jnp.float32),
            pltpu.VMEM((C,), jnp.float32),
            pltpu.VMEM((C,), jnp.float32),
            pltpu.VMEM((C,), jnp.float32),
            pltpu.VMEM((2 * IXB, C), jnp.int32),
            pltpu.VMEM((2 * IXB, C), jnp.int32),
            pltpu.VMEM_SHARED((NPAD, D), jnp.float32),
            pltpu.VMEM_SHARED((NPAD,), jnp.float32),
            pltpu.SemaphoreType.DMA,
            pltpu.SemaphoreType.DMA,
            pltpu.SemaphoreType.DMA,
            pltpu.SemaphoreType.DMA,
        ],
        compiler_params=pltpu.CompilerParams(needs_layout_passes=False),
    )(_edge_body)
    return k(wh, s, t, mx, ei3)


def _combine_body(accp_hbm, denp_hbm, out_hbm, b0, b1, d0, d1):
    cid = lax.axis_index("c")
    sid = lax.axis_index("s")
    wid = sid * NC + cid
    rows = NPAD // NW
    bw = 80
    for j in range(rows // bw):
        base = wid * rows + j * bw
        pltpu.sync_copy(accp_hbm.at[0, pl.ds(base, bw)], b0)
        pltpu.sync_copy(accp_hbm.at[1, pl.ds(base, bw)], b1)
        pltpu.sync_copy(denp_hbm.at[pl.ds(base, bw)], d0)
        pltpu.sync_copy(denp_hbm.at[pl.ds(NPAD + base, bw)], d1)

        def _rbody(r, _):
            dr = 1.0 / (_splat(d0, r) + _splat(d1, r) + 1e-10)
            for i in range(8):
                sl = pl.ds(i * 16, 16)
                b0[r, sl] = (b0[r, sl] + b1[r, sl]) * dr
            return 0
        lax.fori_loop(0, bw, _rbody, 0)
        pltpu.sync_copy(b0, out_hbm.at[pl.ds(base, bw)])


def _combine(accp, denp):
    mesh = plsc.VectorSubcoreMesh(core_axis_name="c", subcore_axis_name="s")
    k = functools.partial(
        pl.kernel,
        out_type=jax.ShapeDtypeStruct((NPAD, D), jnp.float32),
        mesh=mesh,
        scratch_types=[
            pltpu.VMEM((80, D), jnp.float32),
            pltpu.VMEM((80, D), jnp.float32),
            pltpu.VMEM((80,), jnp.float32),
            pltpu.VMEM((80,), jnp.float32),
        ],
        compiler_params=pltpu.CompilerParams(needs_layout_passes=False),
    )(_combine_body)
    return k(accp, denp)


def kernel(h, edge_index, W, a):
    A = (jnp.zeros((8, D), jnp.float32)
         .at[0, :].set(a[:D, 0])
         .at[1, :].set(a[D:, 0]))
    wh, st, mx = _dense(h, W, A)
    ei3 = jnp.pad(edge_index.reshape(2, NCHUNK, C),
                  ((0, 0), (0, CPAD - NCHUNK), (0, 0))).transpose(1, 0, 2)
    ei3 = ei3.reshape(2 * CPAD, C)
    accp, denp = _edge_pass(wh, st[0], st[1], mx, ei3)
    out = _combine(accp, denp)
    return out[:N]


# speedup vs baseline: 22.5456x; 1.1057x over previous
import functools

import jax
import jax.numpy as jnp
from jax import lax
from jax.experimental import pallas as pl
from jax.experimental.pallas import tpu as pltpu
from jax.experimental.pallas import tpu_sc as plsc

N = 10000
E = 320000
D = 128
NEG = 0.2

NC = 2
NS = 16
NW = NC * NS
C = 128
NCHUNK = E // C
KMAX = 80
CPAD = 2576
NPAD = 10240
PER_TILE = NPAD // NS


def _wh_body(h_ref, w_ref, wh_ref):
    wh_ref[...] = jnp.dot(h_ref[...], w_ref[...],
                          preferred_element_type=jnp.float32)


def _st_body(a_ref, wh_ref, st_ref, mx_ref):
    stv = lax.dot_general(
        a_ref[...], wh_ref[...], (((1,), (1,)), ((), ())),
        preferred_element_type=jnp.float32)
    st_ref[...] = stv
    mx_ref[...] = jnp.broadcast_to(jnp.max(stv, axis=1, keepdims=True), (8, D))


def _dense(h, W, A):
    bn = 2000
    wh = pl.pallas_call(
        _wh_body,
        grid=(N // bn,),
        in_specs=[
            pl.BlockSpec((bn, D), lambda i: (i, 0)),
            pl.BlockSpec((D, D), lambda i: (0, 0)),
        ],
        out_specs=pl.BlockSpec((bn, D), lambda i: (i, 0)),
        out_shape=jax.ShapeDtypeStruct((N, D), jnp.float32),
    )(h, W)
    st, mx = pl.pallas_call(
        _st_body,
        in_specs=[
            pl.BlockSpec((8, D), lambda: (0, 0)),
            pl.BlockSpec((N, D), lambda: (0, 0)),
        ],
        out_specs=[
            pl.BlockSpec((8, N), lambda: (0, 0)),
            pl.BlockSpec((8, D), lambda: (0, 0)),
        ],
        out_shape=[
            jax.ShapeDtypeStruct((8, N), jnp.float32),
            jax.ShapeDtypeStruct((8, D), jnp.float32),
        ],
    )(A, wh)
    return wh, st, mx


def _splat(vec_ref, r):
    return plsc.load_gather(vec_ref, [jnp.full((16,), r, jnp.int32)])


IXB = 8
NBATCH = 10


def _edge_body(wh_hbm, s_hbm, t_hbm, mx_hbm, ei_hbm, accp_hbm, denp_hbm,
               rb0, rb1, sb0, sb1, tb0, tb1, xb0, xb1, ixa, ixb,
               acc_sh, den_sh, gsem0, gsem1, ssem0, ssem1):
    cid = lax.axis_index("c")
    sid = lax.axis_index("s")
    wid = sid * NC + cid
    cstart = wid * 80
    cn = jnp.where(wid < NW - 1, 80, NCHUNK - 80 * (NW - 1))
    rbufs, sbufs, tbufs = (rb0, rb1), (sb0, sb1), (tb0, tb1)
    xbufs, ixs = (xb0, xb1), (ixa, ixb)
    gsems, ssems = (gsem0, gsem1), (ssem0, ssem1)

    pltpu.sync_copy(mx_hbm.at[0], sb0)
    pltpu.sync_copy(mx_hbm.at[1], tb0)
    m0 = sb0[pl.ds(0, 16)] + tb0[pl.ds(0, 16)]
    m = jnp.where(m0 > 0, m0, NEG * m0)

    @plsc.parallel_loop(0, C)
    def _zrow(r):
        for i in range(8):
            rb0[r, pl.ds(i * 16, 16)] = jnp.zeros((16,), jnp.float32)
    for i in range(8):
        xb0[pl.ds(i * 16, 16)] = jnp.zeros((16,), jnp.float32)
    base_t = sid * PER_TILE
    def _zacc(j, _):
        off = base_t + j * C
        pltpu.sync_copy(rb0, acc_sh.at[pl.ds(off, C)])
        pltpu.sync_copy(xb0, den_sh.at[pl.ds(off, C)])
        return 0
    lax.fori_loop(0, PER_TILE // C, _zacc, 0)
    plsc.subcore_barrier()

    def _gather(ix, kb, slot):
        pltpu.async_copy(wh_hbm.at[ix.at[2 * kb + 1]], rbufs[slot],
                         gsems[slot])
        pltpu.async_copy(s_hbm.at[ix.at[2 * kb]], sbufs[slot], gsems[slot])
        pltpu.async_copy(t_hbm.at[ix.at[2 * kb + 1]], tbufs[slot],
                         gsems[slot])

    def _gwait(ix, kb, slot):
        pltpu.make_async_copy(wh_hbm.at[ix.at[2 * kb + 1]], rbufs[slot],
                              gsems[slot]).wait()
        pltpu.make_async_copy(s_hbm.at[ix.at[2 * kb]], sbufs[slot],
                              gsems[slot]).wait()
        pltpu.make_async_copy(t_hbm.at[ix.at[2 * kb + 1]], tbufs[slot],
                              gsems[slot]).wait()

    def _swait(ix, kb, slot):
        pltpu.make_async_copy(rbufs[slot], acc_sh.at[ix.at[2 * kb]],
                              ssems[slot]).wait()
        pltpu.make_async_copy(xbufs[slot], den_sh.at[ix.at[2 * kb]],
                              ssems[slot]).wait()

    pltpu.sync_copy(ei_hbm.at[pl.ds(2 * cstart, 2 * IXB)], ixa)
    _gather(ixa, 0, 0)

    def _gbody(gp, _):
        for gg in range(2):
            g = 2 * gp + gg
            ix = ixs[gg]
            for kb in range(IXB):
                k_s = 16 * gp + 8 * gg + kb
                k = g * IXB + kb
                b = kb % 2
                bn = b ^ 1
                if kb == 4:
                    pltpu.sync_copy(
                        ei_hbm.at[pl.ds(2 * (cstart + (g + 1) * IXB),
                                        2 * IXB)],
                        ixs[gg ^ 1])

                @pl.when(k < cn)
                def _():
                    xb, rb = xbufs[b], rbufs[b]
                    _gwait(ix, kb, b)
                    for i in range(C // 16):
                        off = i * 16
                        e = (sbufs[b][pl.ds(off, 16)]
                             + tbufs[b][pl.ds(off, 16)])
                        e = jnp.where(e > 0, e, NEG * e) - m
                        xb[pl.ds(off, 16)] = jnp.exp(e)

                    @pl.when((k + 1 < cn) & (k >= 1))
                    def _():
                        if kb >= 1:
                            _swait(ix, kb - 1, bn)
                        else:
                            _swait(ixs[gg ^ 1], IXB - 1, bn)

                    @pl.when(k + 1 < cn)
                    def _():
                        if kb + 1 < IXB:
                            _gather(ix, kb + 1, bn)
                        else:
                            _gather(ixs[gg ^ 1], 0, bn)

                    @plsc.parallel_loop(0, C, unroll=4)
                    def _scale(r):
                        xr = _splat(xb, r)
                        for i in range(8):
                            sl = pl.ds(i * 16, 16)
                            rb[r, sl] = rb[r, sl] * xr
                    pltpu.async_copy(rb, acc_sh.at[ix.at[2 * kb]], ssems[b],
                                     add=True)
                    pltpu.async_copy(xb, den_sh.at[ix.at[2 * kb]], ssems[b],
                                     add=True)
        return 0
    lax.fori_loop(0, NBATCH // 2, _gbody, 0)
    _swait(ixa, 0, 0)
    _swait(ixa, 0, 1)
    plsc.subcore_barrier()

    pltpu.sync_copy(acc_sh.at[pl.ds(base_t, PER_TILE)],
                    accp_hbm.at[cid, pl.ds(base_t, PER_TILE)])
    pltpu.sync_copy(den_sh.at[pl.ds(base_t, PER_TILE)],
                    denp_hbm.at[pl.ds(cid * NPAD + base_t, PER_TILE)])


def _edge_pass(wh, s, t, mx, ei3):
    mesh = plsc.VectorSubcoreMesh(core_axis_name="c", subcore_axis_name="s")
    k = functools.partial(
        pl.kernel,
        out_type=[
            jax.ShapeDtypeStruct((NC, NPAD, D), jnp.float32),
            jax.ShapeDtypeStruct((NC * NPAD,), jnp.float32),
        ],
        mesh=mesh,
        scratch_types=[
            pltpu.VMEM((C, D), jnp.float32),
            pltpu.VMEM((C, D), jnp.float32),
            pltpu.VMEM((C,), jnp.float32),
            pltpu.VMEM((C,), jnp.float32),
            pltpu.VMEM((C,), jnp.float32),
            pltpu.VMEM((C,), jnp.float32),
            pltpu.VMEM((C,), jnp.float32),
            pltpu.VMEM((C,), jnp.float32),
            pltpu.VMEM((2 * IXB, C), jnp.int32),
            pltpu.VMEM((2 * IXB, C), jnp.int32),
            pltpu.VMEM_SHARED((NPAD, D), jnp.float32),
            pltpu.VMEM_SHARED((NPAD,), jnp.float32),
            pltpu.SemaphoreType.DMA,
            pltpu.SemaphoreType.DMA,
            pltpu.SemaphoreType.DMA,
            pltpu.SemaphoreType.DMA,
        ],
        compiler_params=pltpu.CompilerParams(needs_layout_passes=False),
    )(_edge_body)
    return k(wh, s, t, mx, ei3)


def _combine_body(accp_hbm, denp_hbm, out_hbm, b0, b1, d0, d1, sem):
    cid = lax.axis_index("c")
    sid = lax.axis_index("s")
    wid = sid * NC + cid
    rows = NPAD // NW
    base = wid * rows
    pltpu.async_copy(accp_hbm.at[0, pl.ds(base, rows)], b0, sem)
    pltpu.async_copy(accp_hbm.at[1, pl.ds(base, rows)], b1, sem)
    pltpu.async_copy(denp_hbm.at[pl.ds(base, rows)], d0, sem)
    pltpu.async_copy(denp_hbm.at[pl.ds(NPAD + base, rows)], d1, sem)
    pltpu.make_async_copy(accp_hbm.at[0, pl.ds(base, rows)], b0, sem).wait()
    pltpu.make_async_copy(accp_hbm.at[1, pl.ds(base, rows)], b1, sem).wait()
    pltpu.make_async_copy(denp_hbm.at[pl.ds(base, rows)], d0, sem).wait()
    pltpu.make_async_copy(denp_hbm.at[pl.ds(NPAD + base, rows)], d1,
                          sem).wait()

    for i in range(rows // 16):
        sl = pl.ds(i * 16, 16)
        d0[sl] = 1.0 / (d0[sl] + d1[sl] + 1e-10)

    @plsc.parallel_loop(0, rows, unroll=4)
    def _row(r):
        dr = _splat(d0, r)
        for i in range(8):
            sl = pl.ds(i * 16, 16)
            b0[r, sl] = (b0[r, sl] + b1[r, sl]) * dr
    pltpu.sync_copy(b0, out_hbm.at[pl.ds(base, rows)])


def _combine(accp, denp):
    mesh = plsc.VectorSubcoreMesh(core_axis_name="c", subcore_axis_name="s")
    rows = NPAD // NW
    k = functools.partial(
        pl.kernel,
        out_type=jax.ShapeDtypeStruct((NPAD, D), jnp.float32),
        mesh=mesh,
        scratch_types=[
            pltpu.VMEM((rows, D), jnp.float32),
            pltpu.VMEM((rows, D), jnp.float32),
            pltpu.VMEM((rows,), jnp.float32),
            pltpu.VMEM((rows,), jnp.float32),
            pltpu.SemaphoreType.DMA,
        ],
        compiler_params=pltpu.CompilerParams(needs_layout_passes=False),
    )(_combine_body)
    return k(accp, denp)


def kernel(h, edge_index, W, a):
    A = (jnp.zeros((8, D), jnp.float32)
         .at[0, :].set(a[:D, 0])
         .at[1, :].set(a[D:, 0]))
    wh, st, mx = _dense(h, W, A)
    ei3 = jnp.pad(edge_index.reshape(2, NCHUNK, C),
                  ((0, 0), (0, CPAD - NCHUNK), (0, 0))).transpose(1, 0, 2)
    ei3 = ei3.reshape(2 * CPAD, C)
    accp, denp = _edge_pass(wh, st[0], st[1], mx, ei3)
    out = _combine(accp, denp)
    return out[:N]


# speedup vs baseline: 22.6709x; 1.0056x over previous
import functools

import jax
import jax.numpy as jnp
from jax import lax
from jax.experimental import pallas as pl
from jax.experimental.pallas import tpu as pltpu
from jax.experimental.pallas import tpu_sc as plsc

N = 10000
E = 320000
D = 128
NEG = 0.2

NC = 2
NS = 16
NW = NC * NS
C = 128
NCHUNK = E // C
KMAX = 80
CPAD = 2576
NPAD = 10240
PER_TILE = NPAD // NS


def _wh_body(h_ref, w_ref, wh_ref):
    wh_ref[...] = jnp.dot(h_ref[...], w_ref[...],
                          preferred_element_type=jnp.float32)


def _st_body(a_ref, wh_ref, st_ref, mx_ref):
    stv = lax.dot_general(
        a_ref[...], wh_ref[...], (((1,), (1,)), ((), ())),
        preferred_element_type=jnp.float32)
    st_ref[...] = stv
    mx_ref[...] = jnp.broadcast_to(jnp.max(stv, axis=1, keepdims=True), (8, D))


def _dense(h, W, A):
    bn = 2000
    wh = pl.pallas_call(
        _wh_body,
        grid=(N // bn,),
        in_specs=[
            pl.BlockSpec((bn, D), lambda i: (i, 0)),
            pl.BlockSpec((D, D), lambda i: (0, 0)),
        ],
        out_specs=pl.BlockSpec((bn, D), lambda i: (i, 0)),
        out_shape=jax.ShapeDtypeStruct((N, D), jnp.float32),
    )(h, W)
    st, mx = pl.pallas_call(
        _st_body,
        in_specs=[
            pl.BlockSpec((8, D), lambda: (0, 0)),
            pl.BlockSpec((N, D), lambda: (0, 0)),
        ],
        out_specs=[
            pl.BlockSpec((8, N), lambda: (0, 0)),
            pl.BlockSpec((8, D), lambda: (0, 0)),
        ],
        out_shape=[
            jax.ShapeDtypeStruct((8, N), jnp.float32),
            jax.ShapeDtypeStruct((8, D), jnp.float32),
        ],
    )(A, wh)
    return wh, st, mx


def _splat(vec_ref, r):
    return plsc.load_gather(vec_ref, [jnp.full((16,), r, jnp.int32)])


IXB = 8
NBATCH = 10


def _edge_body(wh_hbm, s_hbm, t_hbm, mx_hbm, ei_hbm, accp_hbm, denp_hbm,
               rb0, rb1, sb0, sb1, tb0, tb1, xb0, xb1, ixa, ixb,
               acc_sh, den_sh, gsem0, gsem1, ssem0, ssem1):
    cid = lax.axis_index("c")
    sid = lax.axis_index("s")
    wid = sid * NC + cid
    cstart = wid * 80
    cn = jnp.where(wid < NW - 1, 80, NCHUNK - 80 * (NW - 1))
    rbufs = (rb0, rb1)
    sbufs, tbufs = (sb0, sb1), (tb0, tb1)
    xbufs, ixs = (xb0, xb1), (ixa, ixb)
    gsems, ssems = (gsem0, gsem1), (ssem0, ssem1)

    pltpu.sync_copy(mx_hbm.at[0], sb0)
    pltpu.sync_copy(mx_hbm.at[1], tb0)
    m0 = sb0[pl.ds(0, 16)] + tb0[pl.ds(0, 16)]
    m = jnp.where(m0 > 0, m0, NEG * m0)

    @plsc.parallel_loop(0, C)
    def _zrow(r):
        for i in range(8):
            rb0[r, pl.ds(i * 16, 16)] = jnp.zeros((16,), jnp.float32)
    for i in range(8):
        xb0[pl.ds(i * 16, 16)] = jnp.zeros((16,), jnp.float32)
    base_t = sid * PER_TILE
    def _zacc(j, _):
        off = base_t + j * C
        pltpu.sync_copy(rb0, acc_sh.at[pl.ds(off, C)])
        pltpu.sync_copy(xb0, den_sh.at[pl.ds(off, C)])
        return 0
    lax.fori_loop(0, PER_TILE // C, _zacc, 0)
    plsc.subcore_barrier()

    def _gather(ix, kb, slot):
        pltpu.async_copy(wh_hbm.at[ix.at[2 * kb + 1]], rbufs[slot],
                         gsems[slot])
        pltpu.async_copy(s_hbm.at[ix.at[2 * kb]], sbufs[slot], gsems[slot])
        pltpu.async_copy(t_hbm.at[ix.at[2 * kb + 1]], tbufs[slot],
                         gsems[slot])

    def _gwait(ix, kb, slot):
        pltpu.make_async_copy(wh_hbm.at[ix.at[2 * kb + 1]], rbufs[slot],
                              gsems[slot]).wait()
        pltpu.make_async_copy(s_hbm.at[ix.at[2 * kb]], sbufs[slot],
                              gsems[slot]).wait()
        pltpu.make_async_copy(t_hbm.at[ix.at[2 * kb + 1]], tbufs[slot],
                              gsems[slot]).wait()

    def _swait(ix, kb, slot):
        pltpu.make_async_copy(rbufs[slot], acc_sh.at[ix.at[2 * kb]],
                              ssems[slot]).wait()
        pltpu.make_async_copy(xbufs[slot], den_sh.at[ix.at[2 * kb]],
                              ssems[slot]).wait()

    pltpu.sync_copy(ei_hbm.at[pl.ds(2 * cstart, 2 * IXB)], ixa)
    _gather(ixa, 0, 0)

    def _gbody(gp, _):
        for gg in range(2):
            g = 2 * gp + gg
            ix = ixs[gg]
            for kb in range(IXB):
                k_s = 16 * gp + 8 * gg + kb
                k = g * IXB + kb
                b = kb % 2
                bn = b ^ 1
                if kb == 4:
                    pltpu.sync_copy(
                        ei_hbm.at[pl.ds(2 * (cstart + (g + 1) * IXB),
                                        2 * IXB)],
                        ixs[gg ^ 1])

                @pl.when(k < cn)
                def _():
                    xb, rb = xbufs[b], rbufs[b]
                    _gwait(ix, kb, b)
                    for i in range(C // 16):
                        off = i * 16
                        e = (sbufs[b][pl.ds(off, 16)]
                             + tbufs[b][pl.ds(off, 16)])
                        e = jnp.where(e > 0, e, NEG * e) - m
                        xb[pl.ds(off, 16)] = jnp.exp(e)

                    @pl.when((k + 1 < cn) & (k >= 1))
                    def _():
                        if kb >= 1:
                            _swait(ix, kb - 1, bn)
                        else:
                            _swait(ixs[gg ^ 1], IXB - 1, bn)

                    @pl.when(k + 1 < cn)
                    def _():
                        if kb + 1 < IXB:
                            _gather(ix, kb + 1, bn)
                        else:
                            _gather(ixs[gg ^ 1], 0, bn)

                    @plsc.parallel_loop(0, C, unroll=4)
                    def _scale(r):
                        xr = _splat(xb, r)
                        for i in range(8):
                            sl = pl.ds(i * 16, 16)
                            rb[r, sl] = rb[r, sl] * xr
                    pltpu.async_copy(rb, acc_sh.at[ix.at[2 * kb]], ssems[b],
                                     add=True)
                    pltpu.async_copy(xb, den_sh.at[ix.at[2 * kb]], ssems[b],
                                     add=True)
        return 0
    lax.fori_loop(0, NBATCH // 2, _gbody, 0)
    _swait(ixa, 0, 0)
    _swait(ixa, 0, 1)
    plsc.subcore_barrier()

    pltpu.sync_copy(acc_sh.at[pl.ds(base_t, PER_TILE)],
                    accp_hbm.at[cid, pl.ds(base_t, PER_TILE)])
    pltpu.sync_copy(den_sh.at[pl.ds(base_t, PER_TILE)],
                    denp_hbm.at[pl.ds(cid * NPAD + base_t, PER_TILE)])


def _edge_pass(wh, s, t, mx, ei3):
    mesh = plsc.VectorSubcoreMesh(core_axis_name="c", subcore_axis_name="s")
    k = functools.partial(
        pl.kernel,
        out_type=[
            jax.ShapeDtypeStruct((NC, NPAD, D), jnp.float32),
            jax.ShapeDtypeStruct((NC * NPAD,), jnp.float32),
        ],
        mesh=mesh,
        scratch_types=[
            pltpu.VMEM((C, D), jnp.float32),
            pltpu.VMEM((C, D), jnp.float32),
            pltpu.VMEM((C,), jnp.float32),
            pltpu.VMEM((C,), jnp.float32),
            pltpu.VMEM((C,), jnp.float32),
            pltpu.VMEM((C,), jnp.float32),
            pltpu.VMEM((C,), jnp.float32),
            pltpu.VMEM((C,), jnp.float32),
            pltpu.VMEM((2 * IXB, C), jnp.int32),
            pltpu.VMEM((2 * IXB, C), jnp.int32),
            pltpu.VMEM_SHARED((NPAD, D), jnp.float32),
            pltpu.VMEM_SHARED((NPAD,), jnp.float32),
            pltpu.SemaphoreType.DMA,
            pltpu.SemaphoreType.DMA,
            pltpu.SemaphoreType.DMA,
            pltpu.SemaphoreType.DMA,
        ],
        compiler_params=pltpu.CompilerParams(needs_layout_passes=False),
    )(_edge_body)
    return k(wh, s, t, mx, ei3)


def _combine_body(accp_hbm, denp_hbm, out_hbm, b0, b1, d0, d1, sem):
    cid = lax.axis_index("c")
    sid = lax.axis_index("s")
    wid = sid * NC + cid
    rows = NPAD // NW
    base = wid * rows
    pltpu.async_copy(accp_hbm.at[0, pl.ds(base, rows)], b0, sem)
    pltpu.async_copy(accp_hbm.at[1, pl.ds(base, rows)], b1, sem)
    pltpu.async_copy(denp_hbm.at[pl.ds(base, rows)], d0, sem)
    pltpu.async_copy(denp_hbm.at[pl.ds(NPAD + base, rows)], d1, sem)
    pltpu.make_async_copy(accp_hbm.at[0, pl.ds(base, rows)], b0, sem).wait()
    pltpu.make_async_copy(accp_hbm.at[1, pl.ds(base, rows)], b1, sem).wait()
    pltpu.make_async_copy(denp_hbm.at[pl.ds(base, rows)], d0, sem).wait()
    pltpu.make_async_copy(denp_hbm.at[pl.ds(NPAD + base, rows)], d1,
                          sem).wait()

    for i in range(rows // 16):
        sl = pl.ds(i * 16, 16)
        d0[sl] = 1.0 / (d0[sl] + d1[sl] + 1e-10)

    @plsc.parallel_loop(0, rows, unroll=4)
    def _row(r):
        dr = _splat(d0, r)
        for i in range(8):
            sl = pl.ds(i * 16, 16)
            b0[r, sl] = (b0[r, sl] + b1[r, sl]) * dr
    pltpu.sync_copy(b0, out_hbm.at[pl.ds(base, rows)])


def _combine(accp, denp):
    mesh = plsc.VectorSubcoreMesh(core_axis_name="c", subcore_axis_name="s")
    rows = NPAD // NW
    k = functools.partial(
        pl.kernel,
        out_type=jax.ShapeDtypeStruct((NPAD, D), jnp.float32),
        mesh=mesh,
        scratch_types=[
            pltpu.VMEM((rows, D), jnp.float32),
            pltpu.VMEM((rows, D), jnp.float32),
            pltpu.VMEM((rows,), jnp.float32),
            pltpu.VMEM((rows,), jnp.float32),
            pltpu.SemaphoreType.DMA,
        ],
        compiler_params=pltpu.CompilerParams(needs_layout_passes=False),
    )(_combine_body)
    return k(accp, denp)


def kernel(h, edge_index, W, a):
    A = (jnp.zeros((8, D), jnp.float32)
         .at[0, :].set(a[:D, 0])
         .at[1, :].set(a[D:, 0]))
    wh, st, mx = _dense(h, W, A)
    ei3 = jnp.pad(edge_index.reshape(2, NCHUNK, C),
                  ((0, 0), (0, CPAD - NCHUNK), (0, 0))).transpose(1, 0, 2)
    ei3 = ei3.reshape(2 * CPAD, C)
    accp, denp = _edge_pass(wh, st[0], st[1], mx, ei3)
    out = _combine(accp, denp)
    return out[:N]
